# baseline (speedup 1.0000x reference)
import functools
import math

import jax
import jax.numpy as jnp
from jax import lax
from jax.experimental import pallas as pl
from jax.experimental.pallas import tpu as pltpu
from jax.experimental.pallas import tpu_sc as plsc

NU = 1024
NI = 4096
NP = 64
K = 20
DELTA = 1e-8
UBLK = 256

_DISCOUNTS = [1.0 / math.log2(r + 2) for r in range(K)]


def _topk_fair_body(scores_ref, rating_ref, pid_ref, expo_ref, recip_ref,
                    rmean_ref, cnt_ref, rsum_ref):
    i = pl.program_id(0)
    scores = scores_ref[...]
    rating = rating_ref[...]
    seen = rating > 0.0
    masked = jnp.where(seen, jnp.float32(-1e10), scores)
    hist = jnp.sum(seen.astype(jnp.float32), axis=1, keepdims=True)
    recip = 1.0 / jnp.maximum(hist, 1.0)
    recip_ref[...] = recip

    @pl.when(i == 0)
    def _():
        rsum_ref[0] = 0.0

    rsum_ref[0] += jnp.sum(recip)

    iota_f = jax.lax.broadcasted_iota(jnp.int32, (UBLK, NI), 1).astype(
        jnp.float32)
    for r in range(K):
        m = jnp.max(masked, axis=1, keepdims=True)
        cand = jnp.where(masked == m, iota_f, jnp.float32(NI))
        idxf = jnp.min(cand, axis=1, keepdims=True)
        sel = cand == idxf
        masked = jnp.where(sel, jnp.float32(-(_DISCOUNTS[r] * 2.0**100)),
                           masked)
    dacc = jnp.where(masked <= -1e20, masked * jnp.float32(-(2.0**-100)),
                     0.0)
    expo = jnp.sum(dacc, axis=0, keepdims=True)

    @pl.when(i == 0)
    def _():
        expo_ref[...] = expo

    @pl.when(i != 0)
    def _():
        expo_ref[...] += expo

    @pl.when(i == pl.num_programs(0) - 1)
    def _():
        rmean_ref[...] = jnp.full((1, 1), 1.0 / NU, jnp.float32) * rsum_ref[0]
        piota = jax.lax.broadcasted_iota(jnp.int32, (NP, NI), 0)
        onehot = pid_ref[...] == piota
        cnt_ref[...] = jnp.sum(onehot.astype(jnp.float32), axis=1,
                               keepdims=True)


def _mm(a, b):
    return jax.lax.dot_general(a, b, (((1,), (0,)), ((), ())),
                               preferred_element_type=jnp.float32)


def _sc_fair_body(expo_hbm, pid_hbm, cnt_hbm, ipw_hbm, pid_v, expo_v, pe_v,
                  cnt_v, ipw_v):
    wid = lax.axis_index("s") * 2 + lax.axis_index("c")

    @pl.when(wid == 0)
    def _():
        pltpu.sync_copy(pid_hbm, pid_v)
        pltpu.sync_copy(expo_hbm, expo_v)
        pltpu.sync_copy(cnt_hbm, cnt_v)
        for k in range(NP // 16):
            pe_v[pl.ds(k * 16, 16)] = jnp.zeros((16,), jnp.float32)

        def hist_body(k, carry):
            idx = pid_v[pl.ds(k * 16, 16)]
            e = expo_v[pl.ds(k * 16, 16)]
            plsc.addupdate_scatter(pe_v, [idx], e)
            return carry

        lax.fori_loop(0, NI // 16, hist_body, 0)

        ones = jnp.full((16,), 1.0, jnp.float32)
        acc = jnp.zeros((16,), jnp.float32)
        for k in range(NP // 16):
            pe = pe_v[pl.ds(k * 16, 16)]
            cnt = cnt_v[pl.ds(k * 16, 16)]
            pf = ones / jnp.maximum(pe / jnp.maximum(cnt, ones) + DELTA,
                                    DELTA)
            pe_v[pl.ds(k * 16, 16)] = pf
            acc = acc + pf
        pf_mean = jnp.broadcast_to(jnp.sum(acc), (16,)) * (1.0 / NP)
        for k in range(NP // 16):
            pe_v[pl.ds(k * 16, 16)] = pe_v[pl.ds(k * 16, 16)] / pf_mean

        def gather_body(k, a):
            idx = pid_v[pl.ds(k * 16, 16)]
            w = plsc.load_gather(pe_v, [idx])
            ipw_v[pl.ds(k * 16, 16)] = w
            return a + w

        acc2 = lax.fori_loop(0, NI // 16, gather_body,
                             jnp.zeros((16,), jnp.float32))
        ipw_mean = jnp.broadcast_to(jnp.sum(acc2), (16,)) * (1.0 / NI)

        def norm_body(k, carry):
            ipw_v[pl.ds(k * 16, 16)] = ipw_v[pl.ds(k * 16, 16)] / ipw_mean
            return carry

        lax.fori_loop(0, NI // 16, norm_body, 0)
        pltpu.sync_copy(ipw_v, ipw_hbm)


_sc_fair = functools.partial(
    pl.kernel,
    mesh=plsc.VectorSubcoreMesh(core_axis_name="c", subcore_axis_name="s"),
    out_type=jax.ShapeDtypeStruct((NI,), jnp.float32),
    compiler_params=pltpu.CompilerParams(needs_layout_passes=False),
    scratch_types=[
        pltpu.VMEM((NI,), jnp.int32),
        pltpu.VMEM((NI,), jnp.float32),
        pltpu.VMEM((NP,), jnp.float32),
        pltpu.VMEM((NP,), jnp.float32),
        pltpu.VMEM((NI,), jnp.float32),
    ],
)(_sc_fair_body)


def _call_topk(scores, rating_matrix, pid2):
    nblk = NU // UBLK
    return pl.pallas_call(
        _topk_fair_body,
        grid=(nblk,),
        in_specs=[
            pl.BlockSpec((UBLK, NI), lambda i: (i, 0)),
            pl.BlockSpec((UBLK, NI), lambda i: (i, 0)),
            pl.BlockSpec((1, NI), lambda i: (0, 0)),
        ],
        out_specs=[
            pl.BlockSpec((1, NI), lambda i: (0, 0)),
            pl.BlockSpec((UBLK, 1), lambda i: (i, 0)),
            pl.BlockSpec((1, 1), lambda i: (0, 0)),
            pl.BlockSpec((NP, 1), lambda i: (0, 0)),
        ],
        out_shape=[
            jax.ShapeDtypeStruct((1, NI), jnp.float32),
            jax.ShapeDtypeStruct((NU, 1), jnp.float32),
            jax.ShapeDtypeStruct((1, 1), jnp.float32),
            jax.ShapeDtypeStruct((NP, 1), jnp.float32),
        ],
        scratch_shapes=[pltpu.SMEM((1,), jnp.float32)],
    )(scores, rating_matrix, pid2)


def _mlp_loss_body(x_ref, eW1_ref, eb1_ref, eW2_ref, eb2_ref, eW3_ref,
                   eb3_ref, pW1_ref, pb1_ref, pW2_ref, pb2_ref, uW1_ref,
                   ub1_ref, uW2_ref, ub2_ref, ipw_ref, recip_ref, rmean_ref,
                   loss_ref, uf_ref):
    i = pl.program_id(0)
    relu = jax.nn.relu
    x = x_ref[...]
    uf = recip_ref[...] / rmean_ref[0, 0]
    uf_ref[...] = uf
    h1 = relu(_mm(x, eW1_ref[...]) + eb1_ref[...])
    h2 = relu(_mm(h1, eW2_ref[...]) + eb2_ref[...])
    h_enc = (_mm(h2, eW3_ref[...]) + eb3_ref[...]) * x

    pW1, pb1, pW2, pb2 = pW1_ref[...], pb1_ref[...], pW2_ref[...], pb2_ref[...]
    uW1, ub1, uW2, ub2 = uW1_ref[...], ub1_ref[...], uW2_ref[...], ub2_ref[...]

    h_p = _mm(relu(_mm(h_enc, pW1) + pb1), pW2) + pb2
    t_p = _mm(relu(_mm(ipw_ref[...] * x, pW1) + pb1), pW2) + pb2
    h_u = _mm(relu(_mm(h_enc, uW1) + ub1), uW2) + ub2
    t_u = _mm(relu(_mm(uf * x, uW1) + ub1), uW2) + ub2

    blk = (jnp.sum((h_p - t_p) ** 2) + jnp.sum((h_u - t_u) ** 2)) / (NU * NI)
    blk = jnp.full((1, 1), 1.0, jnp.float32) * blk

    @pl.when(i == 0)
    def _():
        loss_ref[...] = blk

    @pl.when(i != 0)
    def _():
        loss_ref[...] += blk


def kernel(scores, rating_matrix, eW1, eb1, eW2, eb2, eW3, eb3, pW1, pb1,
           pW2, pb2, uW1, ub1, uW2, ub2, provider_ids):
    nblk = NU // UBLK
    pid2 = provider_ids.reshape(1, NI).astype(jnp.int32)
    expo, recip, rmean, cnt = _call_topk(scores, rating_matrix, pid2)

    ipw_flat = _sc_fair(expo.reshape(NI), provider_ids.astype(jnp.int32),
                        cnt.reshape(NP))
    ipw = ipw_flat.reshape(1, NI)

    full = lambda shape: pl.BlockSpec(shape, lambda i: (0,) * len(shape))
    loss, uf = pl.pallas_call(
        _mlp_loss_body,
        grid=(nblk,),
        in_specs=[
            pl.BlockSpec((UBLK, NI), lambda i: (i, 0)),
            full((NI, 256)), full((1, 256)),
            full((256, 128)), full((1, 128)),
            full((128, NI)), full((1, NI)),
            full((NI, 128)), full((1, 128)),
            full((128, NI)), full((1, NI)),
            full((NI, 128)), full((1, 128)),
            full((128, NI)), full((1, NI)),
            full((1, NI)),
            pl.BlockSpec((UBLK, 1), lambda i: (i, 0)),
            full((1, 1)),
        ],
        out_specs=[
            pl.BlockSpec((1, 1), lambda i: (0, 0)),
            pl.BlockSpec((UBLK, 1), lambda i: (i, 0)),
        ],
        out_shape=[
            jax.ShapeDtypeStruct((1, 1), jnp.float32),
            jax.ShapeDtypeStruct((NU, 1), jnp.float32),
        ],
    )(rating_matrix, eW1, eb1.reshape(1, -1), eW2, eb2.reshape(1, -1),
      eW3, eb3.reshape(1, -1), pW1, pb1.reshape(1, -1), pW2,
      pb2.reshape(1, -1), uW1, ub1.reshape(1, -1), uW2, ub2.reshape(1, -1),
      ipw, recip, rmean)

    return (loss.reshape(()), ipw.reshape(NI), uf.reshape(NU),
            expo.reshape(NI))


# speedup vs baseline: 1.0245x; 1.0245x over previous
import functools
import math

import jax
import jax.numpy as jnp
from jax import lax
from jax.experimental import pallas as pl
from jax.experimental.pallas import tpu as pltpu
from jax.experimental.pallas import tpu_sc as plsc

NU = 1024
NI = 4096
NP = 64
K = 20
DELTA = 1e-8
UBLK = 256

_DISCOUNTS = [1.0 / math.log2(r + 2) for r in range(K)]


def _topk_fair_body(scores_ref, rating_ref, expo_ref, recip_ref,
                    rmean_ref, rsum_ref):
    i = pl.program_id(0)
    scores = scores_ref[...]
    rating = rating_ref[...]
    seen = rating > 0.0
    masked = jnp.where(seen, jnp.float32(-1e10), scores)
    hist = jnp.sum(seen.astype(jnp.float32), axis=1, keepdims=True)
    recip = 1.0 / jnp.maximum(hist, 1.0)
    recip_ref[...] = recip

    @pl.when(i == 0)
    def _():
        rsum_ref[0] = 0.0

    rsum_ref[0] += jnp.sum(recip)

    iota_f = jax.lax.broadcasted_iota(jnp.int32, (UBLK, NI), 1).astype(
        jnp.float32)
    for r in range(K):
        m = jnp.max(masked, axis=1, keepdims=True)
        cand = jnp.where(masked == m, iota_f, jnp.float32(NI))
        idxf = jnp.min(cand, axis=1, keepdims=True)
        sel = cand == idxf
        masked = jnp.where(sel, jnp.float32(-(_DISCOUNTS[r] * 2.0**100)),
                           masked)
    dacc = jnp.where(masked <= -1e20, masked * jnp.float32(-(2.0**-100)),
                     0.0)
    expo = jnp.sum(dacc, axis=0, keepdims=True)

    @pl.when(i == 0)
    def _():
        expo_ref[...] = expo

    @pl.when(i != 0)
    def _():
        expo_ref[...] += expo

    @pl.when(i == pl.num_programs(0) - 1)
    def _():
        rmean_ref[...] = jnp.full((1, 1), 1.0 / NU, jnp.float32) * rsum_ref[0]


def _mm(a, b):
    return jax.lax.dot_general(a, b, (((1,), (0,)), ((), ())),
                               preferred_element_type=jnp.float32)


def _sc_fair_body(expo_hbm, pid_hbm, ipw_hbm, pid_v, expo_v, pe_v, cnt_v,
                  ipw_v):
    wid = lax.axis_index("s") * 2 + lax.axis_index("c")

    @pl.when(wid == 0)
    def _():
        pltpu.sync_copy(pid_hbm, pid_v)
        pltpu.sync_copy(expo_hbm, expo_v)
        for k in range(NP // 16):
            pe_v[pl.ds(k * 16, 16)] = jnp.zeros((16,), jnp.float32)
            cnt_v[pl.ds(k * 16, 16)] = jnp.zeros((16,), jnp.float32)

        def hist_body(k, carry):
            idx = pid_v[pl.ds(k * 16, 16)]
            e = expo_v[pl.ds(k * 16, 16)]
            plsc.addupdate_scatter(pe_v, [idx], e)
            plsc.addupdate_scatter(cnt_v, [idx],
                                   jnp.full((16,), 1.0, jnp.float32))
            return carry

        lax.fori_loop(0, NI // 16, hist_body, 0)

        ones = jnp.full((16,), 1.0, jnp.float32)
        acc = jnp.zeros((16,), jnp.float32)
        for k in range(NP // 16):
            pe = pe_v[pl.ds(k * 16, 16)]
            cnt = cnt_v[pl.ds(k * 16, 16)]
            pf = ones / jnp.maximum(pe / jnp.maximum(cnt, ones) + DELTA,
                                    DELTA)
            pe_v[pl.ds(k * 16, 16)] = pf
            acc = acc + pf
        pf_mean = jnp.broadcast_to(jnp.sum(acc), (16,)) * (1.0 / NP)
        acc2 = jnp.zeros((16,), jnp.float32)
        for k in range(NP // 16):
            pf_n = pe_v[pl.ds(k * 16, 16)] / pf_mean
            pe_v[pl.ds(k * 16, 16)] = pf_n
            acc2 = acc2 + pf_n * cnt_v[pl.ds(k * 16, 16)]
        ipw_mean = jnp.broadcast_to(jnp.sum(acc2), (16,)) * (1.0 / NI)
        inv = ones / ipw_mean

        def gather_body(k, carry):
            idx = pid_v[pl.ds(k * 16, 16)]
            w = plsc.load_gather(pe_v, [idx])
            ipw_v[pl.ds(k * 16, 16)] = w * inv
            return carry

        lax.fori_loop(0, NI // 16, gather_body, 0)
        pltpu.sync_copy(ipw_v, ipw_hbm)


_sc_fair = functools.partial(
    pl.kernel,
    mesh=plsc.VectorSubcoreMesh(core_axis_name="c", subcore_axis_name="s"),
    out_type=jax.ShapeDtypeStruct((NI,), jnp.float32),
    compiler_params=pltpu.CompilerParams(needs_layout_passes=False),
    scratch_types=[
        pltpu.VMEM((NI,), jnp.int32),
        pltpu.VMEM((NI,), jnp.float32),
        pltpu.VMEM((NP,), jnp.float32),
        pltpu.VMEM((NP,), jnp.float32),
        pltpu.VMEM((NI,), jnp.float32),
    ],
)(_sc_fair_body)


def _mlp_loss_body(x_ref, eW1_ref, eb1_ref, eW2_ref, eb2_ref, eW3_ref,
                   eb3_ref, pW1_ref, pb1_ref, pW2_ref, pb2_ref, uW1_ref,
                   ub1_ref, uW2_ref, ub2_ref, ipw_ref, recip_ref, rmean_ref,
                   loss_ref, uf_ref):
    i = pl.program_id(0)
    relu = jax.nn.relu
    x = x_ref[...]
    uf = recip_ref[...] / rmean_ref[0, 0]
    uf_ref[...] = uf
    h1 = relu(_mm(x, eW1_ref[...]) + eb1_ref[...])
    h2 = relu(_mm(h1, eW2_ref[...]) + eb2_ref[...])
    h_enc = (_mm(h2, eW3_ref[...]) + eb3_ref[...]) * x

    pW1, pb1, pW2, pb2 = pW1_ref[...], pb1_ref[...], pW2_ref[...], pb2_ref[...]
    uW1, ub1, uW2, ub2 = uW1_ref[...], ub1_ref[...], uW2_ref[...], ub2_ref[...]

    h_p = _mm(relu(_mm(h_enc, pW1) + pb1), pW2) + pb2
    t_p = _mm(relu(_mm(ipw_ref[...] * x, pW1) + pb1), pW2) + pb2
    h_u = _mm(relu(_mm(h_enc, uW1) + ub1), uW2) + ub2
    t_u = _mm(relu(_mm(uf * x, uW1) + ub1), uW2) + ub2

    blk = (jnp.sum((h_p - t_p) ** 2) + jnp.sum((h_u - t_u) ** 2)) / (NU * NI)
    blk = jnp.full((1, 1), 1.0, jnp.float32) * blk

    @pl.when(i == 0)
    def _():
        loss_ref[...] = blk

    @pl.when(i != 0)
    def _():
        loss_ref[...] += blk


def kernel(scores, rating_matrix, eW1, eb1, eW2, eb2, eW3, eb3, pW1, pb1,
           pW2, pb2, uW1, ub1, uW2, ub2, provider_ids):
    nblk = NU // UBLK
    expo, recip, rmean = pl.pallas_call(
        _topk_fair_body,
        grid=(nblk,),
        in_specs=[
            pl.BlockSpec((UBLK, NI), lambda i: (i, 0)),
            pl.BlockSpec((UBLK, NI), lambda i: (i, 0)),
        ],
        out_specs=[
            pl.BlockSpec((1, NI), lambda i: (0, 0)),
            pl.BlockSpec((UBLK, 1), lambda i: (i, 0)),
            pl.BlockSpec((1, 1), lambda i: (0, 0)),
        ],
        out_shape=[
            jax.ShapeDtypeStruct((1, NI), jnp.float32),
            jax.ShapeDtypeStruct((NU, 1), jnp.float32),
            jax.ShapeDtypeStruct((1, 1), jnp.float32),
        ],
        scratch_shapes=[pltpu.SMEM((1,), jnp.float32)],
    )(scores, rating_matrix)

    ipw_flat = _sc_fair(expo.reshape(NI), provider_ids.astype(jnp.int32))
    ipw = ipw_flat.reshape(1, NI)

    full = lambda shape: pl.BlockSpec(shape, lambda i: (0,) * len(shape))
    loss, uf = pl.pallas_call(
        _mlp_loss_body,
        grid=(nblk,),
        in_specs=[
            pl.BlockSpec((UBLK, NI), lambda i: (i, 0)),
            full((NI, 256)), full((1, 256)),
            full((256, 128)), full((1, 128)),
            full((128, NI)), full((1, NI)),
            full((NI, 128)), full((1, 128)),
            full((128, NI)), full((1, NI)),
            full((NI, 128)), full((1, 128)),
            full((128, NI)), full((1, NI)),
            full((1, NI)),
            pl.BlockSpec((UBLK, 1), lambda i: (i, 0)),
            full((1, 1)),
        ],
        out_specs=[
            pl.BlockSpec((1, 1), lambda i: (0, 0)),
            pl.BlockSpec((UBLK, 1), lambda i: (i, 0)),
        ],
        out_shape=[
            jax.ShapeDtypeStruct((1, 1), jnp.float32),
            jax.ShapeDtypeStruct((NU, 1), jnp.float32),
        ],
    )(rating_matrix, eW1, eb1.reshape(1, -1), eW2, eb2.reshape(1, -1),
      eW3, eb3.reshape(1, -1), pW1, pb1.reshape(1, -1), pW2,
      pb2.reshape(1, -1), uW1, ub1.reshape(1, -1), uW2, ub2.reshape(1, -1),
      ipw, recip, rmean)

    return (loss.reshape(()), ipw.reshape(NI), uf.reshape(NU),
            expo.reshape(NI))
